# baseline (speedup 1.0000x reference)
import functools

import jax
import jax.numpy as jnp
from jax import lax
from jax.experimental import pallas as pl
from jax.experimental.pallas import tpu as pltpu
from jax.experimental.pallas import tpu_sc as plsc

_NP = 5120
_NW = 32
_CPW = _NP // _NW
_SC_C0 = 4608
_SCW = (_NP - _SC_C0) // _NW
_RV = _NP // 16
_SIGMA = 0.5

_BJ = 512
_BI = 512


_BJS = 512
_BIL = 512
_NCH = _NP // _BIL


_BIG = 3.0e38


def _tc_body(featC_ref, featT_ref, out_ref):
    p = pl.program_id(0)
    jx1 = jnp.broadcast_to(featT_ref[:, 0:1], (_BJ, _BIL))
    jy1 = jnp.broadcast_to(featT_ref[:, 1:2], (_BJ, _BIL))
    jx2 = jnp.broadcast_to(featT_ref[:, 2:3], (_BJ, _BIL))
    jy2 = jnp.broadcast_to(featT_ref[:, 3:4], (_BJ, _BIL))
    js_col = featT_ref[:, 4:5]
    jsup_col = lax.bitcast_convert_type(
        lax.bitcast_convert_type(js_col, jnp.int32) + 1, jnp.float32)
    jsb = jnp.broadcast_to(js_col, (_BJ, _BIL))
    jsup = jnp.broadcast_to(jsup_col, (_BJ, _BIL))
    ja_col = (featT_ref[:, 2:3] - featT_ref[:, 0:1]) * (
        featT_ref[:, 3:4] - featT_ref[:, 1:2])
    jab = jnp.broadcast_to(ja_col, (_BJ, _BIL))
    ilota = lax.broadcasted_iota(jnp.int32, (1, _BIL), 1)
    jiota = p * _BJ + lax.broadcasted_iota(jnp.int32, (_BJ, 1), 0)

    def chunk(r, acc):
        r0 = r * _BIL
        thr = jnp.where((r0 + ilota) < jiota, jsb, jsup)
        x1 = featC_ref[0:1, pl.ds(r0, _BIL)]
        y1 = featC_ref[1:2, pl.ds(r0, _BIL)]
        x2 = featC_ref[2:3, pl.ds(r0, _BIL)]
        y2 = featC_ref[3:4, pl.ds(r0, _BIL)]
        sv = featC_ref[4:5, pl.ds(r0, _BIL)]
        iw = jnp.maximum(jnp.minimum(x2, jx2) - jnp.maximum(x1, jx1), 0.0)
        ih = jnp.minimum(y2, jy2) - jnp.maximum(y1, jy1)
        inter = jnp.maximum(iw * ih, 1e-20)
        asum = ((x2 - x1) * (y2 - y1) + jab)
        ratio = asum / inter
        return jnp.minimum(acc, jnp.where(sv >= thr, ratio, _BIG))

    acc0 = jnp.full((_BJ, _BIL), _BIG, jnp.float32)
    acc1 = jnp.full((_BJ, _BIL), _BIG, jnp.float32)
    for r in range(0, _NCH, 2):
        acc0 = chunk(r, acc0)
        acc1 = chunk(r + 1, acc1)
    mr = jnp.min(jnp.minimum(acc0, acc1), axis=1, keepdims=True)
    m = 1.0 / (mr - 1.0)
    out_ref[...] = js_col * jnp.exp(m * m * (-1.0 / _SIGMA))


def _matrix_nms_tc(featC, featT, n_cols):
    return pl.pallas_call(
        _tc_body,
        grid=(n_cols // _BJ,),
        in_specs=[
            pl.BlockSpec((8, _NP), lambda j: (0, 0)),
            pl.BlockSpec((_BJ, 8), lambda j: (j, 0)),
        ],
        out_specs=pl.BlockSpec((_BJ, 1), lambda j: (j, 0)),
        out_shape=jax.ShapeDtypeStruct((n_cols, 1), jnp.float32),
    )(featC, featT)

_mesh = plsc.VectorSubcoreMesh(core_axis_name="c", subcore_axis_name="s")


@functools.partial(
    pl.kernel,
    mesh=_mesh,
    out_type=jax.ShapeDtypeStruct((_NP - _SC_C0,), jnp.float32),
    scratch_types=[
        pltpu.VMEM((6, _NP), jnp.float32),
        pltpu.VMEM((_SCW,), jnp.float32),
    ],
)
def _matrix_nms_sc(feat_hbm, out_hbm, feat, outv):
    cid = lax.axis_index("c")
    sid = lax.axis_index("s")
    wid = sid * 2 + cid
    base = _SC_C0 + wid * _SCW

    pltpu.sync_copy(feat_hbm, feat)

    def bcast(v, k):
        return jnp.full((16,), v[k], dtype=jnp.float32)

    def group_body(g, carry):
        gb = base + g * 16
        x1c = feat[0, pl.ds(gb, 16)]
        y1c = feat[1, pl.ds(gb, 16)]
        x2c = feat[2, pl.ds(gb, 16)]
        y2c = feat[3, pl.ds(gb, 16)]
        scc = feat[4, pl.ds(gb, 16)]
        fc = feat[5, pl.ds(gb, 16)]
        areac = (x2c - x1c) * (y2c - y1c)

        def rows_body(r, acc):
            o = r * 16
            x1 = feat[0, pl.ds(o, 16)]
            y1 = feat[1, pl.ds(o, 16)]
            x2 = feat[2, pl.ds(o, 16)]
            y2 = feat[3, pl.ds(o, 16)]
            sv = feat[4, pl.ds(o, 16)]
            fv = feat[5, pl.ds(o, 16)]
            areav = (x2 - x1) * (y2 - y1)
            for k in range(16):
                bx1 = bcast(x1, k)
                by1 = bcast(y1, k)
                bx2 = bcast(x2, k)
                by2 = bcast(y2, k)
                bs = bcast(sv, k)
                bf = bcast(fv, k)
                ba = bcast(areav, k)
                iw = jnp.maximum(
                    jnp.minimum(bx2, x2c) - jnp.maximum(bx1, x1c), 0.0)
                ih = jnp.maximum(
                    jnp.minimum(by2, y2c) - jnp.maximum(by1, y1c), 0.0)
                inter = iw * ih
                union = (ba + areac) - inter
                iou = inter / union
                keep = (bs > scc) | ((bs == scc) & (bf < fc))
                acc = jnp.maximum(acc, jnp.where(keep, iou, 0.0))
            return acc

        acc = lax.fori_loop(0, _RV, rows_body, jnp.zeros((16,), jnp.float32))
        outv[pl.ds(g * 16, 16)] = scc * jnp.exp(acc * acc * (-1.0 / _SIGMA))
        return carry

    lax.fori_loop(0, _SCW // 16, group_body, 0)
    pltpu.sync_copy(outv, out_hbm.at[pl.ds(wid * _SCW, _SCW)])


def kernel(boxes, scores):
    b = boxes.astype(jnp.float32)
    s = scores.astype(jnp.float32)
    n = s.shape[0]
    pad = _NP - n
    x1 = jnp.concatenate([b[:, 0], jnp.zeros((pad,), jnp.float32)])
    y1 = jnp.concatenate([b[:, 1], jnp.zeros((pad,), jnp.float32)])
    x2 = jnp.concatenate([b[:, 2], jnp.ones((pad,), jnp.float32)])
    y2 = jnp.concatenate([b[:, 3], jnp.ones((pad,), jnp.float32)])
    sc = jnp.concatenate([s, jnp.full((pad,), -1.0, jnp.float32)])
    idxf = jnp.arange(_NP, dtype=jnp.float32)
    z = jnp.zeros((_NP,), jnp.float32)
    featC = jnp.stack([x1, y1, x2, y2, sc, idxf, z, z])
    featT = featC.T
    out_tc = _matrix_nms_tc(featC, featT, _SC_C0)
    feat6 = jnp.stack([x1, y1, x2, y2, sc, idxf])
    out_sc = _matrix_nms_sc(feat6)
    return jnp.concatenate([out_tc[:, 0], out_sc])[:n]


# speedup vs baseline: 1.3472x; 1.3472x over previous
import jax
import jax.numpy as jnp
from jax import lax
from jax.experimental import pallas as pl

_NP = 5120
_SIGMA = 0.5
_BJ = 512
_BIL = 512
_NCH = _NP // _BIL
_BIG = 3.0e38


def _tc_body(featC_ref, featT_ref, out_ref):
    p = pl.program_id(0)
    jx1 = jnp.broadcast_to(featT_ref[:, 0:1], (_BJ, _BIL))
    jy1 = jnp.broadcast_to(featT_ref[:, 1:2], (_BJ, _BIL))
    jx2 = jnp.broadcast_to(featT_ref[:, 2:3], (_BJ, _BIL))
    jy2 = jnp.broadcast_to(featT_ref[:, 3:4], (_BJ, _BIL))
    js_col = featT_ref[:, 4:5]
    jsup_col = lax.bitcast_convert_type(
        lax.bitcast_convert_type(js_col, jnp.int32) + 1, jnp.float32)
    jsb = jnp.broadcast_to(js_col, (_BJ, _BIL))
    jsup = jnp.broadcast_to(jsup_col, (_BJ, _BIL))
    ja_col = (featT_ref[:, 2:3] - featT_ref[:, 0:1]) * (
        featT_ref[:, 3:4] - featT_ref[:, 1:2])
    jab = jnp.broadcast_to(ja_col, (_BJ, _BIL))
    ilota = lax.broadcasted_iota(jnp.int32, (1, _BIL), 1)
    jiota = p * _BJ + lax.broadcasted_iota(jnp.int32, (_BJ, 1), 0)

    def chunk(r, acc):
        r0 = r * _BIL
        thr = jnp.where((r0 + ilota) < jiota, jsb, jsup)
        x1 = featC_ref[0:1, pl.ds(r0, _BIL)]
        y1 = featC_ref[1:2, pl.ds(r0, _BIL)]
        x2 = featC_ref[2:3, pl.ds(r0, _BIL)]
        y2 = featC_ref[3:4, pl.ds(r0, _BIL)]
        sv = featC_ref[4:5, pl.ds(r0, _BIL)]
        iw = jnp.maximum(jnp.minimum(x2, jx2) - jnp.maximum(x1, jx1), 0.0)
        ih = jnp.minimum(y2, jy2) - jnp.maximum(y1, jy1)
        inter = jnp.maximum(iw * ih, 1e-20)
        asum = ((x2 - x1) * (y2 - y1) + jab)
        ratio = asum / inter
        return jnp.minimum(acc, jnp.where(sv >= thr, ratio, _BIG))

    acc0 = jnp.full((_BJ, _BIL), _BIG, jnp.float32)
    acc1 = jnp.full((_BJ, _BIL), _BIG, jnp.float32)
    for r in range(0, _NCH, 2):
        acc0 = chunk(r, acc0)
        acc1 = chunk(r + 1, acc1)
    mr = jnp.min(jnp.minimum(acc0, acc1), axis=1, keepdims=True)
    m = 1.0 / (mr - 1.0)
    out_ref[...] = js_col * jnp.exp(m * m * (-1.0 / _SIGMA))


def _matrix_nms_tc(featC, featT, n_cols):
    return pl.pallas_call(
        _tc_body,
        grid=(n_cols // _BJ,),
        in_specs=[
            pl.BlockSpec((8, _NP), lambda j: (0, 0)),
            pl.BlockSpec((_BJ, 8), lambda j: (j, 0)),
        ],
        out_specs=pl.BlockSpec((_BJ, 1), lambda j: (j, 0)),
        out_shape=jax.ShapeDtypeStruct((n_cols, 1), jnp.float32),
    )(featC, featT)


def kernel(boxes, scores):
    b = boxes.astype(jnp.float32)
    s = scores.astype(jnp.float32)
    n = s.shape[0]
    pad = _NP - n
    x1 = jnp.concatenate([b[:, 0], jnp.zeros((pad,), jnp.float32)])
    y1 = jnp.concatenate([b[:, 1], jnp.zeros((pad,), jnp.float32)])
    x2 = jnp.concatenate([b[:, 2], jnp.ones((pad,), jnp.float32)])
    y2 = jnp.concatenate([b[:, 3], jnp.ones((pad,), jnp.float32)])
    sc = jnp.concatenate([s, jnp.full((pad,), -1.0, jnp.float32)])
    z = jnp.zeros((_NP,), jnp.float32)
    featC = jnp.stack([x1, y1, x2, y2, sc, z, z, z])
    featT = featC.T
    out_tc = _matrix_nms_tc(featC, featT, _NP)
    return out_tc[:n, 0]


# speedup vs baseline: 1.3518x; 1.0034x over previous
import jax
import jax.numpy as jnp
from jax import lax
from jax.experimental import pallas as pl

_NP = 5120
_SIGMA = 0.5
_BJ = 1024
_BIL = 512
_NCH = _NP // _BIL
_BIG = 3.0e38


def _tc_body(featC_ref, featT_ref, out_ref):
    p = pl.program_id(0)
    jx1 = jnp.broadcast_to(featT_ref[:, 0:1], (_BJ, _BIL))
    jy1 = jnp.broadcast_to(featT_ref[:, 1:2], (_BJ, _BIL))
    jx2 = jnp.broadcast_to(featT_ref[:, 2:3], (_BJ, _BIL))
    jy2 = jnp.broadcast_to(featT_ref[:, 3:4], (_BJ, _BIL))
    js_col = featT_ref[:, 4:5]
    jsup_col = lax.bitcast_convert_type(
        lax.bitcast_convert_type(js_col, jnp.int32) + 1, jnp.float32)
    jsb = jnp.broadcast_to(js_col, (_BJ, _BIL))
    jsup = jnp.broadcast_to(jsup_col, (_BJ, _BIL))
    ja_col = (featT_ref[:, 2:3] - featT_ref[:, 0:1]) * (
        featT_ref[:, 3:4] - featT_ref[:, 1:2])
    jab = jnp.broadcast_to(ja_col, (_BJ, _BIL))
    ilota = lax.broadcasted_iota(jnp.int32, (1, _BIL), 1)
    jiota = p * _BJ + lax.broadcasted_iota(jnp.int32, (_BJ, 1), 0)

    def chunk(r, acc):
        r0 = r * _BIL
        thr = jnp.where((r0 + ilota) < jiota, jsb, jsup)
        x1 = featC_ref[0:1, pl.ds(r0, _BIL)]
        y1 = featC_ref[1:2, pl.ds(r0, _BIL)]
        x2 = featC_ref[2:3, pl.ds(r0, _BIL)]
        y2 = featC_ref[3:4, pl.ds(r0, _BIL)]
        sv = featC_ref[4:5, pl.ds(r0, _BIL)]
        iw = jnp.maximum(jnp.minimum(x2, jx2) - jnp.maximum(x1, jx1), 0.0)
        ih = jnp.minimum(y2, jy2) - jnp.maximum(y1, jy1)
        inter = jnp.maximum(iw * ih, 1e-20)
        asum = ((x2 - x1) * (y2 - y1) + jab)
        ratio = asum / inter
        return jnp.minimum(acc, jnp.where(sv >= thr, ratio, _BIG))

    acc0 = jnp.full((_BJ, _BIL), _BIG, jnp.float32)
    acc1 = jnp.full((_BJ, _BIL), _BIG, jnp.float32)
    for r in range(0, _NCH, 2):
        acc0 = chunk(r, acc0)
        acc1 = chunk(r + 1, acc1)
    mr = jnp.min(jnp.minimum(acc0, acc1), axis=1, keepdims=True)
    m = 1.0 / (mr - 1.0)
    out_ref[...] = js_col * jnp.exp(m * m * (-1.0 / _SIGMA))


def _matrix_nms_tc(featC, featT, n_cols):
    return pl.pallas_call(
        _tc_body,
        grid=(n_cols // _BJ,),
        in_specs=[
            pl.BlockSpec((8, _NP), lambda j: (0, 0)),
            pl.BlockSpec((_BJ, 8), lambda j: (j, 0)),
        ],
        out_specs=pl.BlockSpec((_BJ, 1), lambda j: (j, 0)),
        out_shape=jax.ShapeDtypeStruct((n_cols, 1), jnp.float32),
    )(featC, featT)


def kernel(boxes, scores):
    b = boxes.astype(jnp.float32)
    s = scores.astype(jnp.float32)
    n = s.shape[0]
    pad = _NP - n
    x1 = jnp.concatenate([b[:, 0], jnp.zeros((pad,), jnp.float32)])
    y1 = jnp.concatenate([b[:, 1], jnp.zeros((pad,), jnp.float32)])
    x2 = jnp.concatenate([b[:, 2], jnp.ones((pad,), jnp.float32)])
    y2 = jnp.concatenate([b[:, 3], jnp.ones((pad,), jnp.float32)])
    sc = jnp.concatenate([s, jnp.full((pad,), -1.0, jnp.float32)])
    z = jnp.zeros((_NP,), jnp.float32)
    featC = jnp.stack([x1, y1, x2, y2, sc, z, z, z])
    featT = featC.T
    out_tc = _matrix_nms_tc(featC, featT, _NP)
    return out_tc[:n, 0]
